# Initial kernel scaffold; baseline (speedup 1.0000x reference)
import jax
import jax.numpy as jnp
from jax.experimental import pallas as pl


def kernel(users, pos_items, neg_items, user_embeds, item_embeds, beta_uD, beta_iD, ii_neighbor_mat, ii_constraint_mat):
    raise NotImplementedError("write your pallas kernel here")


# speedup vs baseline: 3.4677x; 3.4677x over previous
import functools

import jax
import jax.numpy as jnp
from jax import lax
from jax.experimental import pallas as pl
from jax.experimental.pallas import tpu as pltpu
from jax.experimental.pallas import tpu_sc as plsc

USER_N = 100000
ITEM_N = 100000
EMB = 64
B = 16384
NNEG = 20
KNN = 10
GAMMA_C = 1e-05
LAMBDA_C = 1e-05

NC = 2
NS = 16
NW = NC * NS
BPW = B // NW
C = 32
NCH = BPW // C
NEG_ROWS = C * NNEG // 128

_mesh = plsc.VectorSubcoreMesh(core_axis_name="c", subcore_axis_name="s")


def _sc_body(users4, pos4, neg4, uemb, iemb, buD, biD, iinbr, iicon,
             posp_o, negp_o, ip_o, bu_o, bip_o, bin_o, sim_o,
             userc_v, posc_v, negidx_v, nbr_v, ue_v, pe_v, ne_v, nb_v,
             posp_v, negp_v, ip_v, bu_b, bip_b, bin_b, sim_b, sem):
    wid = lax.axis_index("s") * NC + lax.axis_index("c")

    def chunk_body(c, carry):
        pltpu.sync_copy(users4.at[wid, c], userc_v)
        pltpu.sync_copy(pos4.at[wid, c], posc_v)
        pltpu.sync_copy(neg4.at[wid, c], negidx_v)
        cps = []
        cps.append(pltpu.async_copy(uemb.at[userc_v], ue_v, sem))
        cps.append(pltpu.async_copy(iemb.at[posc_v], pe_v, sem))
        for j in range(NEG_ROWS):
            cps.append(pltpu.async_copy(
                iemb.at[negidx_v.at[j]], ne_v.at[pl.ds(j * 128, 128)], sem))
            cps.append(pltpu.async_copy(
                biD.at[negidx_v.at[j]],
                bin_b.at[pl.ds(c * C * NNEG + j * 128, 128)], sem))
        cps.append(pltpu.async_copy(buD.at[userc_v],
                                    bu_b.at[pl.ds(c * C, C)], sem))
        cps.append(pltpu.async_copy(biD.at[posc_v],
                                    bip_b.at[pl.ds(c * C, C)], sem))
        for cp in cps:
            cp.wait()
        lanecap = jnp.minimum(lax.iota(jnp.int32, 16), KNN - 1)
        for h in range(2):
            pv16 = posc_v[pl.ds(h * 16, 16)]
            cps2 = []
            for g in range(16):
                u = h * 16 + g
                pu = jnp.broadcast_to(pv16[g], (16,))
                cps2.append(pltpu.async_copy(
                    iinbr.at[pu * KNN + lanecap], nbr_v.at[u], sem))
                cps2.append(pltpu.async_copy(
                    iicon.at[pu * KNN + lanecap], sim_b.at[c * C + u], sem))
            for cp in cps2:
                cp.wait()
        for h in range(2):
            cps2 = []
            for g in range(16):
                u = h * 16 + g
                idxv = nbr_v[u, :]
                cps2.append(pltpu.async_copy(
                    iemb.at[idxv], nb_v.at[pl.ds(u * 16, 16)], sem))
            for cp in cps2:
                cp.wait()

        def user_body(u, carry2):
            uek = [ue_v[u, pl.ds(k * 16, 16)] for k in range(EMB // 16)]

            def dot16(rref, r):
                acc = uek[0] * rref[r, pl.ds(0, 16)]
                for k in range(1, EMB // 16):
                    acc = acc + uek[k] * rref[r, pl.ds(k * 16, 16)]
                return acc

            posp_v[u, :] = dot16(pe_v, u)

            def negj(j, carry3):
                negp_v[u * NNEG + j, :] = dot16(ne_v, u * NNEG + j)
                return carry3
            lax.fori_loop(0, NNEG, negj, 0)

            def nbrj(j, carry3):
                ip_v[u * KNN + j, :] = dot16(nb_v, u * 16 + j)
                return carry3
            lax.fori_loop(0, KNN, nbrj, 0)
            return carry2
        lax.fori_loop(0, C, user_body, 0)

        row0 = wid * BPW + c * C
        pltpu.sync_copy(posp_v, posp_o.at[pl.ds(row0, C)])
        pltpu.sync_copy(negp_v, negp_o.at[pl.ds(row0 * NNEG, C * NNEG)])
        pltpu.sync_copy(ip_v, ip_o.at[pl.ds(row0 * KNN, C * KNN)])
        return carry
    lax.fori_loop(0, NCH, chunk_body, 0)

    base = wid * BPW
    pltpu.sync_copy(bu_b, bu_o.at[pl.ds(base, BPW)])
    pltpu.sync_copy(bip_b, bip_o.at[pl.ds(base, BPW)])
    pltpu.sync_copy(bin_b, bin_o.at[pl.ds(base * NNEG, BPW * NNEG)])
    pltpu.sync_copy(sim_b, sim_o.at[pl.ds(base, BPW)])


_sc_scores = functools.partial(
    pl.kernel,
    mesh=_mesh,
    compiler_params=pltpu.CompilerParams(use_tc_tiling_on_sc=False),
    out_type=[
        jax.ShapeDtypeStruct((B, 16), jnp.float32),
        jax.ShapeDtypeStruct((B * NNEG, 16), jnp.float32),
        jax.ShapeDtypeStruct((B * KNN, 16), jnp.float32),
        jax.ShapeDtypeStruct((B,), jnp.float32),
        jax.ShapeDtypeStruct((B,), jnp.float32),
        jax.ShapeDtypeStruct((B * NNEG,), jnp.float32),
        jax.ShapeDtypeStruct((B, 16), jnp.float32),
    ],
    scratch_types=[
        pltpu.VMEM((C,), jnp.int32),
        pltpu.VMEM((C,), jnp.int32),
        pltpu.VMEM((NEG_ROWS, 128), jnp.int32),
        pltpu.VMEM((C, 16), jnp.int32),
        pltpu.VMEM((C, EMB), jnp.float32),
        pltpu.VMEM((C, EMB), jnp.float32),
        pltpu.VMEM((C * NNEG, EMB), jnp.float32),
        pltpu.VMEM((C * 16, EMB), jnp.float32),
        pltpu.VMEM((C, 16), jnp.float32),
        pltpu.VMEM((C * NNEG, 16), jnp.float32),
        pltpu.VMEM((C * KNN, 16), jnp.float32),
        pltpu.VMEM((BPW,), jnp.float32),
        pltpu.VMEM((BPW,), jnp.float32),
        pltpu.VMEM((BPW * NNEG,), jnp.float32),
        pltpu.VMEM((BPW, 16), jnp.float32),
        pltpu.SemaphoreType.DMA,
    ],
)(_sc_body)


ROWS2 = USER_N * EMB // 128
G = 64
TST = 10
TROWS = ROWS2 // TST
SB = B // G


def _tc_body(posp_r, negp_r, ip_r, bu_r, bip_r, bin_r, sim_r, ue_r, ie_r,
             out_r):
    i = pl.program_id(0)

    @pl.when(i == 0)
    def _init():
        out_r[0, 0] = jnp.float32(0.0)

    @pl.when(i < TST)
    def _norm():
        ue = ue_r[...]
        ie = ie_r[...]
        out_r[0, 0] += GAMMA_C * 0.5 * (jnp.sum(ue * ue) + jnp.sum(ie * ie))

    def sp(x):
        return jnp.maximum(x, 0.0) + jnp.log1p(jnp.exp(-jnp.abs(x)))

    pos_s = jnp.sum(posp_r[...], axis=1)
    neg_s = jnp.sum(negp_r[...], axis=1).reshape(SB, NNEG)
    i_s = jnp.sum(ip_r[...], axis=1).reshape(SB, KNN)
    bu = bu_r[...]
    pw = 1.0 + bu[:, 0] * bip_r[...][:, 0]
    pos_part = jnp.sum(pw * sp(-pos_s))
    nw = 1.0 + bu * bin_r[...]
    neg_part = jnp.sum(nw * sp(neg_s)) * (1.0 / NNEG)
    i_part = jnp.sum(sim_r[...][:, :KNN] * sp(-i_s))
    out_r[0, 0] += pos_part + neg_part + LAMBDA_C * i_part


def _tmap(i):
    return (jnp.minimum(i, TST - 1), 0)


_tc_loss = pl.pallas_call(
    _tc_body,
    grid=(G,),
    in_specs=[
        pl.BlockSpec((SB, 16), lambda i: (i, 0)),
        pl.BlockSpec((SB * NNEG, 16), lambda i: (i, 0)),
        pl.BlockSpec((SB * KNN, 16), lambda i: (i, 0)),
        pl.BlockSpec((SB, 1), lambda i: (i, 0)),
        pl.BlockSpec((SB, 1), lambda i: (i, 0)),
        pl.BlockSpec((SB, NNEG), lambda i: (i, 0)),
        pl.BlockSpec((SB, 16), lambda i: (i, 0)),
        pl.BlockSpec((TROWS, 128), _tmap),
        pl.BlockSpec((TROWS, 128), _tmap),
    ],
    out_specs=pl.BlockSpec((1, 1), lambda i: (0, 0), memory_space=pltpu.SMEM),
    out_shape=jax.ShapeDtypeStruct((1, 1), jnp.float32),
)


def kernel(users, pos_items, neg_items, user_embeds, item_embeds, beta_uD,
           beta_iD, ii_neighbor_mat, ii_constraint_mat):
    users4 = users.reshape(NW, NCH, C).astype(jnp.int32)
    pos4 = pos_items.reshape(NW, NCH, C).astype(jnp.int32)
    neg4 = neg_items.reshape(NW, NCH, NEG_ROWS, 128).astype(jnp.int32)
    posp, negp, ip, bu, bip, bin_, sim = _sc_scores(
        users4, pos4, neg4, user_embeds, item_embeds, beta_uD, beta_iD,
        ii_neighbor_mat.astype(jnp.int32).reshape(-1),
        ii_constraint_mat.reshape(-1))
    out = _tc_loss(
        posp, negp, ip,
        bu.reshape(B, 1), bip.reshape(B, 1), bin_.reshape(B, NNEG), sim,
        user_embeds.reshape(ROWS2, 128), item_embeds.reshape(ROWS2, 128))
    return out[0, 0]


# speedup vs baseline: 7.1945x; 2.0747x over previous
import functools

import jax
import jax.numpy as jnp
from jax import lax
from jax.experimental import pallas as pl
from jax.experimental.pallas import tpu as pltpu
from jax.experimental.pallas import tpu_sc as plsc

USER_N = 100000
ITEM_N = 100000
EMB = 64
B = 16384
NNEG = 20
KNN = 10
GAMMA_C = 1e-05
LAMBDA_C = 1e-05

NC = 2
NS = 16
NW = NC * NS
BPW = B // NW
C = 32
NCH = BPW // C
NEG_ROWS = C * NNEG // 128
KNN_ROWS = C * KNN // 64

_mesh = plsc.VectorSubcoreMesh(core_axis_name="c", subcore_axis_name="s")


def _sc_body(users4, pos4, neg4, nsidx4, buidx4, uemb, iemb, buD, biD,
             iinbr, iicon,
             pos_o, neg_o, i_o, bu_o, bip_o, bin_o, burep_o, simrep_o,
             userc_v, posc_v, negidx_v, nsidx_v, buidx_v, nbrid_v,
             ue_v, pe_v, ne_v, nb_v,
             pos_sb, neg_sb, i_sb, bu_b, bip_b, bin_b, burep_b, simf_b, sem):
    wid = lax.axis_index("s") * NC + lax.axis_index("c")
    lanes = lax.iota(jnp.int32, 16)
    p8 = lanes ^ 8
    p4 = lanes ^ 4
    p2 = lanes ^ 2
    p1 = lanes ^ 1
    lmasks = [lanes == l for l in range(16)]

    def hsum(acc):
        acc = acc + acc.at[p8].get(mode="promise_in_bounds")
        acc = acc + acc.at[p4].get(mode="promise_in_bounds")
        acc = acc + acc.at[p2].get(mode="promise_in_bounds")
        acc = acc + acc.at[p1].get(mode="promise_in_bounds")
        return acc

    def chunk_body(c, carry):
        pltpu.sync_copy(users4.at[wid, c], userc_v)
        pltpu.sync_copy(pos4.at[wid, c], posc_v)
        pltpu.sync_copy(neg4.at[wid, c], negidx_v)
        pltpu.sync_copy(nsidx4.at[wid, c], nsidx_v)
        pltpu.sync_copy(buidx4.at[wid, c], buidx_v)
        cps = []
        cps.append(pltpu.async_copy(uemb.at[userc_v], ue_v, sem))
        cps.append(pltpu.async_copy(iemb.at[posc_v], pe_v, sem))
        for j in range(NEG_ROWS):
            cps.append(pltpu.async_copy(
                iemb.at[negidx_v.at[j]], ne_v.at[pl.ds(j * 128, 128)], sem))
            cps.append(pltpu.async_copy(
                biD.at[negidx_v.at[j]],
                bin_b.at[pl.ds(c * C * NNEG + j * 128, 128)], sem))
            cps.append(pltpu.async_copy(
                buD.at[buidx_v.at[j]],
                burep_b.at[pl.ds(c * C * NNEG + j * 128, 128)], sem))
        for j in range(KNN_ROWS):
            cps.append(pltpu.async_copy(
                iinbr.at[nsidx_v.at[j]], nbrid_v.at[j], sem))
            cps.append(pltpu.async_copy(
                iicon.at[nsidx_v.at[j]],
                simf_b.at[pl.ds(c * C * KNN + j * 64, 64)], sem))
        cps.append(pltpu.async_copy(buD.at[userc_v],
                                    bu_b.at[pl.ds(c * C, C)], sem))
        cps.append(pltpu.async_copy(biD.at[posc_v],
                                    bip_b.at[pl.ds(c * C, C)], sem))
        for cp in cps:
            cp.wait()
        cps = []
        for j in range(KNN_ROWS):
            cps.append(pltpu.async_copy(
                iemb.at[nbrid_v.at[j]], nb_v.at[pl.ds(j * 64, 64)], sem))
        for cp in cps:
            cp.wait()

        def uek_of(u):
            return [ue_v[u, pl.ds(k * 16, 16)] for k in range(EMB // 16)]

        def dot16(uek, rref, r):
            acc = uek[0] * rref[r, pl.ds(0, 16)]
            for k in range(1, EMB // 16):
                acc = acc + uek[k] * rref[r, pl.ds(k * 16, 16)]
            return hsum(acc)

        def posg(g, carry2):
            d0 = g * 16
            vec = jnp.zeros((16,), jnp.float32)
            for l in range(16):
                d = d0 + l
                s = dot16(uek_of(d), pe_v, d)
                vec = jnp.where(lmasks[l], s, vec)
            pos_sb[pl.ds(c * C + d0, 16)] = vec
            return carry2
        lax.fori_loop(0, C // 16, posg, 0)

        def negg(g, carry2):
            d0 = g * 16
            vec = jnp.zeros((16,), jnp.float32)
            for l in range(16):
                d = d0 + l
                s = dot16(uek_of(lax.div(d, NNEG)), ne_v, d)
                vec = jnp.where(lmasks[l], s, vec)
            neg_sb[pl.ds(c * C * NNEG + d0, 16)] = vec
            return carry2
        lax.fori_loop(0, C * NNEG // 16, negg, 0)

        def nbrg(g, carry2):
            d0 = g * 16
            vec = jnp.zeros((16,), jnp.float32)
            for l in range(16):
                d = d0 + l
                s = dot16(uek_of(lax.div(d, KNN)), nb_v, d)
                vec = jnp.where(lmasks[l], s, vec)
            i_sb[pl.ds(c * C * KNN + d0, 16)] = vec
            return carry2
        lax.fori_loop(0, C * KNN // 16, nbrg, 0)
        return carry
    lax.fori_loop(0, NCH, chunk_body, 0)

    base = wid * BPW
    pltpu.sync_copy(pos_sb, pos_o.at[pl.ds(base, BPW)])
    pltpu.sync_copy(neg_sb, neg_o.at[pl.ds(base * NNEG, BPW * NNEG)])
    pltpu.sync_copy(i_sb, i_o.at[pl.ds(base * KNN, BPW * KNN)])
    pltpu.sync_copy(bu_b, bu_o.at[pl.ds(base, BPW)])
    pltpu.sync_copy(bip_b, bip_o.at[pl.ds(base, BPW)])
    pltpu.sync_copy(bin_b, bin_o.at[pl.ds(base * NNEG, BPW * NNEG)])
    pltpu.sync_copy(burep_b, burep_o.at[pl.ds(base * NNEG, BPW * NNEG)])
    pltpu.sync_copy(simf_b, simrep_o.at[pl.ds(base * KNN, BPW * KNN)])


_sc_scores = functools.partial(
    pl.kernel,
    mesh=_mesh,
    compiler_params=pltpu.CompilerParams(use_tc_tiling_on_sc=False),
    out_type=[
        jax.ShapeDtypeStruct((B,), jnp.float32),
        jax.ShapeDtypeStruct((B * NNEG,), jnp.float32),
        jax.ShapeDtypeStruct((B * KNN,), jnp.float32),
        jax.ShapeDtypeStruct((B,), jnp.float32),
        jax.ShapeDtypeStruct((B,), jnp.float32),
        jax.ShapeDtypeStruct((B * NNEG,), jnp.float32),
        jax.ShapeDtypeStruct((B * NNEG,), jnp.float32),
        jax.ShapeDtypeStruct((B * KNN,), jnp.float32),
    ],
    scratch_types=[
        pltpu.VMEM((C,), jnp.int32),
        pltpu.VMEM((C,), jnp.int32),
        pltpu.VMEM((NEG_ROWS, 128), jnp.int32),
        pltpu.VMEM((KNN_ROWS, 64), jnp.int32),
        pltpu.VMEM((NEG_ROWS, 128), jnp.int32),
        pltpu.VMEM((KNN_ROWS, 64), jnp.int32),
        pltpu.VMEM((C, EMB), jnp.float32),
        pltpu.VMEM((C, EMB), jnp.float32),
        pltpu.VMEM((C * NNEG, EMB), jnp.float32),
        pltpu.VMEM((C * KNN, EMB), jnp.float32),
        pltpu.VMEM((BPW,), jnp.float32),
        pltpu.VMEM((BPW * NNEG,), jnp.float32),
        pltpu.VMEM((BPW * KNN,), jnp.float32),
        pltpu.VMEM((BPW,), jnp.float32),
        pltpu.VMEM((BPW,), jnp.float32),
        pltpu.VMEM((BPW * NNEG,), jnp.float32),
        pltpu.VMEM((BPW * NNEG,), jnp.float32),
        pltpu.VMEM((BPW * KNN,), jnp.float32),
        pltpu.SemaphoreType.DMA,
    ],
)(_sc_body)


ROWS2 = USER_N * EMB // 128
G = 10
TROWS = ROWS2 // G


def _tc_body(pos_r, bu_r, bip_r, neg_r, bin_r, burep_r, is_r, simrep_r,
             ue_r, ie_r, out_r):
    i = pl.program_id(0)

    @pl.when(i == 0)
    def _init():
        out_r[0, 0] = jnp.float32(0.0)

    ue = ue_r[...]
    ie = ie_r[...]
    out_r[0, 0] += GAMMA_C * 0.5 * (jnp.sum(ue * ue) + jnp.sum(ie * ie))

    @pl.when(i == 0)
    def _scores():
        def sp(x):
            return jnp.maximum(x, 0.0) + jnp.log1p(jnp.exp(-jnp.abs(x)))

        pw = 1.0 + bu_r[...] * bip_r[...]
        pos_part = jnp.sum(pw * sp(-pos_r[...]))
        nw = 1.0 + burep_r[...] * bin_r[...]
        neg_part = jnp.sum(nw * sp(neg_r[...])) * (1.0 / NNEG)
        i_part = jnp.sum(simrep_r[...] * sp(-is_r[...]))
        out_r[0, 0] += pos_part + neg_part + LAMBDA_C * i_part


def _cmap(i):
    return (0, 0)


_tc_loss = pl.pallas_call(
    _tc_body,
    grid=(G,),
    in_specs=[
        pl.BlockSpec((B // 128, 128), _cmap),
        pl.BlockSpec((B // 128, 128), _cmap),
        pl.BlockSpec((B // 128, 128), _cmap),
        pl.BlockSpec((B * NNEG // 128, 128), _cmap),
        pl.BlockSpec((B * NNEG // 128, 128), _cmap),
        pl.BlockSpec((B * NNEG // 128, 128), _cmap),
        pl.BlockSpec((B * KNN // 128, 128), _cmap),
        pl.BlockSpec((B * KNN // 128, 128), _cmap),
        pl.BlockSpec((TROWS, 128), lambda i: (i, 0)),
        pl.BlockSpec((TROWS, 128), lambda i: (i, 0)),
    ],
    out_specs=pl.BlockSpec((1, 1), lambda i: (0, 0), memory_space=pltpu.SMEM),
    out_shape=jax.ShapeDtypeStruct((1, 1), jnp.float32),
)


def kernel(users, pos_items, neg_items, user_embeds, item_embeds, beta_uD,
           beta_iD, ii_neighbor_mat, ii_constraint_mat):
    users32 = users.astype(jnp.int32)
    pos32 = pos_items.astype(jnp.int32)
    users4 = users32.reshape(NW, NCH, C)
    pos4 = pos32.reshape(NW, NCH, C)
    neg4 = neg_items.astype(jnp.int32).reshape(NW, NCH, NEG_ROWS, 128)
    nsidx4 = (pos32[:, None] * KNN + jnp.arange(KNN, dtype=jnp.int32)
              ).reshape(NW, NCH, KNN_ROWS, 64)
    buidx4 = jnp.repeat(users32, NNEG).reshape(NW, NCH, NEG_ROWS, 128)
    pos_s, neg_s, i_s, bu, bip, bin_, burep, simrep = _sc_scores(
        users4, pos4, neg4, nsidx4, buidx4, user_embeds, item_embeds,
        beta_uD, beta_iD,
        ii_neighbor_mat.astype(jnp.int32).reshape(-1),
        ii_constraint_mat.reshape(-1))
    out = _tc_loss(
        pos_s.reshape(B // 128, 128),
        bu.reshape(B // 128, 128),
        bip.reshape(B // 128, 128),
        neg_s.reshape(B * NNEG // 128, 128),
        bin_.reshape(B * NNEG // 128, 128),
        burep.reshape(B * NNEG // 128, 128),
        i_s.reshape(B * KNN // 128, 128),
        simrep.reshape(B * KNN // 128, 128),
        user_embeds.reshape(ROWS2, 128), item_embeds.reshape(ROWS2, 128))
    return out[0, 0]
